# baseline (speedup 1.0000x reference)
import functools

import jax
import jax.numpy as jnp
from jax.experimental import pallas as pl


def _mm_body(a_ref, b_ref, o_ref):
    a_blk = a_ref[0].astype(jnp.bfloat16)
    b_blk = b_ref[0].astype(jnp.bfloat16)
    o_ref[0] = jax.lax.dot_general(
        a_blk, b_blk,
        dimension_numbers=(((1,), (0,)), ((), ())),
        preferred_element_type=jnp.float32,
    )


@functools.partial(jax.jit, static_argnames=("tm",))
def _batched_mm(a3, b3, tm=512):
    nb, m, k = a3.shape
    n = b3.shape[-1]
    return pl.pallas_call(
        _mm_body,
        grid=(nb, m // tm),
        in_specs=[
            pl.BlockSpec((1, tm, k), lambda b, i: (b, i, 0)),
            pl.BlockSpec((1, k, n), lambda b, i: (b, 0, 0)),
        ],
        out_specs=pl.BlockSpec((1, tm, n), lambda b, i: (b, i, 0)),
        out_shape=jax.ShapeDtypeStruct((nb, m, n), jnp.float32),
    )(a3, b3)


def kernel(a, b):
    B1, B2, M, K = a.shape
    N = b.shape[-1]
    a3 = a.reshape(B1 * B2, M, K)
    b3 = b.reshape(B1 * B2, K, N)
    out = _batched_mm(a3, b3, tm=min(2048, M))
    return out.reshape(B1, B2, M, N)


# speedup vs baseline: 1.0215x; 1.0215x over previous
import functools

import jax
import jax.numpy as jnp
from jax.experimental import pallas as pl


def _mm_body(a_ref, b_ref, o_ref):
    a_blk = a_ref[0].astype(jnp.bfloat16)
    b_blk = b_ref[0].astype(jnp.bfloat16)
    o_ref[0] = jax.lax.dot_general(
        a_blk, b_blk,
        dimension_numbers=(((1,), (0,)), ((), ())),
        preferred_element_type=jnp.float32,
    )


@functools.partial(jax.jit, static_argnames=("tm",))
def _batched_mm(a3, b3, tm=512):
    nb, m, k = a3.shape
    n = b3.shape[-1]
    return pl.pallas_call(
        _mm_body,
        grid=(nb, m // tm),
        in_specs=[
            pl.BlockSpec((1, tm, k), lambda b, i: (b, i, 0)),
            pl.BlockSpec((1, k, n), lambda b, i: (b, 0, 0)),
        ],
        out_specs=pl.BlockSpec((1, tm, n), lambda b, i: (b, i, 0)),
        out_shape=jax.ShapeDtypeStruct((nb, m, n), jnp.float32),
    )(a3, b3)


def kernel(a, b):
    B1, B2, M, K = a.shape
    N = b.shape[-1]
    a3 = a.reshape(B1 * B2, M, K)
    b3 = b.reshape(B1 * B2, K, N)
    out = _batched_mm(a3, b3, tm=min(1024, M))
    return out.reshape(B1, B2, M, N)
